# baseline (speedup 1.0000x reference)
import functools

import jax
import jax.numpy as jnp
from jax.experimental import pallas as pl
from jax.experimental.pallas import tpu as pltpu

_EPS = 1e-8
_BLK = 128

_DW, _DB, _PW, _PB, _UW, _UB = 0, 4, 8, 12, 16, 20
_BW, _BB, _EW0, _EW1, _EB = 24, 25, 26, 27, 28


def _unet_kernel(sc_ref, gs_ref, sw_ref, out_ref,
                 u1, u2, u3, u4, yv, cv, sv, ov, mv, *, n_real, ks):
    uscr = [u1, u2, u3, u4]
    f32, bf16 = jnp.float32, jnp.bfloat16
    n = gs_ref.shape[1]
    nb = n // _BLK

    def loop(body):
        jax.lax.fori_loop(0, nb, lambda j, c: (body(j * _BLK), 0)[1], 0)

    def mv_gs(h):
        def body(r):
            yv[pl.ds(r, _BLK), :] = jnp.dot(
                gs_ref[0, pl.ds(r, _BLK), :], h, preferred_element_type=f32)
        loop(body)
        return yv[...]

    def mv_u(uref, h):
        hi = h.astype(bf16)
        r1 = h - hi.astype(f32)
        mid = r1.astype(bf16)
        lo = (r1 - mid.astype(f32)).astype(bf16)
        hh = jnp.concatenate([hi, mid, lo], axis=1)

        def body(r):
            y3 = jnp.dot(uref[pl.ds(r, _BLK), :], hh,
                         preferred_element_type=f32)
            yv[pl.ds(r, _BLK), :] = jnp.sum(y3, axis=1, keepdims=True)
        loop(body)
        return yv[...]

    iota = jax.lax.broadcasted_iota(jnp.int32, (n, 1), 0)
    m = (iota < n_real).astype(f32)
    o = iota.astype(f32)

    h = jnp.maximum(mv_gs(sw_ref[...]), 0.0) * m
    org_h = h

    down_outs, saved, rs = [], [], None
    for i in range(4):
        if i == 0:
            y = mv_gs(h)
        else:
            y = mv_u(uscr[i - 1], h) / (rs + _EPS)
        h = jnp.maximum(y * sc_ref[0, _DW + i] + sc_ref[0, _DB + i], 0.0) * m
        down_outs.append(h)
        saved.append((None if i == 0 else i - 1, rs, m))

        s = jax.nn.sigmoid(h * sc_ref[0, _PW + i] + sc_ref[0, _PB + i])
        sv[...], ov[...], mv[...] = s, o, m
        s_r, o_r, m_r = jnp.transpose(s), jnp.transpose(o), jnp.transpose(m)

        def rank_body(r):
            sb = sv[pl.ds(r, _BLK), :]
            ob = ov[pl.ds(r, _BLK), :]
            better = (s_r > sb) | ((s_r == sb) & (o_r < ob))
            cv[pl.ds(r, _BLK), :] = jnp.sum(
                jnp.where(better, m_r, 0.0), axis=1, keepdims=True)
        loop(rank_body)
        cnt = cv[...]
        newm = m * (cnt < ks[i]).astype(f32)
        o = cnt
        h = h * s * newm

        if i == 0:
            rhs = (gs_ref[0] != 0).astype(bf16)
        else:
            rhs = uscr[i - 1][...]
        mv[...] = newm
        newm_rb = jnp.transpose(newm).astype(bf16)
        udst = uscr[i]
        one_b = jnp.ones((n, 1), bf16)

        def sq_body(r, lvl=i, rhs=rhs, newm_rb=newm_rb, udst=udst,
                    one_b=one_b):
            if lvl == 0:
                lhs = (gs_ref[0, pl.ds(r, _BLK), :] != 0).astype(bf16)
            else:
                lhs = uscr[lvl - 1][pl.ds(r, _BLK), :]
            v = jnp.dot(lhs, rhs, preferred_element_type=f32)
            vb = v.astype(bf16)
            sel = jnp.where(vb > 0, jnp.array(1, bf16), jnp.array(0, bf16))
            bmat = sel * mv[pl.ds(r, _BLK), :].astype(bf16) * newm_rb
            udst[pl.ds(r, _BLK), :] = bmat
            cv[pl.ds(r, _BLK), :] = jnp.dot(bmat, one_b,
                                            preferred_element_type=f32)
        loop(sq_body)
        rs = cv[...]
        m = newm

    y = mv_u(uscr[3], h) / (rs + _EPS)
    h = jnp.maximum(y * sc_ref[0, _BW] + sc_ref[0, _BB], 0.0) * m

    for i in range(4):
        up = 3 - i
        lvl, rsu, mu = saved[up]
        if lvl is None:
            y = mv_gs(h)
        else:
            y = mv_u(uscr[lvl], h) / (rsu + _EPS)
        h = jnp.maximum(y * sc_ref[0, _UW + i] + sc_ref[0, _UB + i], 0.0) * mu
        h = h + down_outs[up]

    y1 = mv_gs(h)
    y2 = mv_gs(org_h)
    out_ref[0] = jnp.maximum(
        y1 * sc_ref[0, _EW0] + y2 * sc_ref[0, _EW1] + sc_ref[0, _EB], 0.0)


def _mlp_kernel(x_ref, g1, b1, w1, c1, g2, b2, w2, c2, g3, b3, w3, c3,
                g4, b4, w4, c4, out_ref):
    def bn(x, ga, be):
        mu = jnp.mean(x, axis=0, keepdims=True)
        va = jnp.mean((x - mu) * (x - mu), axis=0, keepdims=True)
        return (x - mu) / jnp.sqrt(va + 1e-5) * ga[...] + be[...]

    f32 = jnp.float32
    h = x_ref[...]
    for ga, be, w, c in ((g1, b1, w1, c1), (g2, b2, w2, c2),
                         (g3, b3, w3, c3), (g4, b4, w4, c4)):
        h = jnp.maximum(bn(h, ga, be), 0.0)
        h = jnp.dot(h, w[...], preferred_element_type=f32) + c[...]
    zmax = jnp.max(h, axis=1, keepdims=True)
    e = jnp.exp(h - zmax)
    out_ref[...] = e / jnp.sum(e, axis=1, keepdims=True)


def _pad_to(x, shape):
    return jnp.pad(x, [(0, t - s) for s, t in zip(x.shape, shape)])


@jax.jit
def kernel(g, params):
    p = params
    b, n_real, _ = g.shape
    n = ((n_real + 127) // 128) * 128
    ks = [int(n_real * 4 / 5), int(n_real * 3 / 5),
          int(n_real * 2 / 5), int(n_real * 1 / 5)]

    gp = _pad_to(g.astype(jnp.float32), (b, n, n))
    sw = _pad_to(p['start_w'], (n, 1))

    sc = jnp.zeros((32,), jnp.float32)
    for i in range(4):
        sc = sc.at[_DW + i].set(p['down_w'][i][0, 0])
        sc = sc.at[_DB + i].set(p['down_b'][i][0])
        sc = sc.at[_PW + i].set(p['pool_w'][i][0, 0])
        sc = sc.at[_PB + i].set(p['pool_b'][i][0])
        sc = sc.at[_UW + i].set(p['up_w'][i][0, 0])
        sc = sc.at[_UB + i].set(p['up_b'][i][0])
    sc = sc.at[_BW].set(p['bottom_w'][0, 0]).at[_BB].set(p['bottom_b'][0])
    sc = sc.at[_EW0].set(p['end_w'][0, 0]).at[_EW1].set(p['end_w'][1, 0])
    sc = sc.at[_EB].set(p['end_b'][0])
    sc = sc.reshape(1, 32)

    vec = lambda: pltpu.VMEM((n, 1), jnp.float32)
    feats = pl.pallas_call(
        functools.partial(_unet_kernel, n_real=n_real, ks=ks),
        grid=(b,),
        in_specs=[
            pl.BlockSpec((1, 32), lambda i: (0, 0), memory_space=pltpu.SMEM),
            pl.BlockSpec((1, n, n), lambda i: (i, 0, 0)),
            pl.BlockSpec((n, 1), lambda i: (0, 0)),
        ],
        out_specs=pl.BlockSpec((1, n, 1), lambda i: (i, 0, 0)),
        out_shape=jax.ShapeDtypeStruct((b, n, 1), jnp.float32),
        scratch_shapes=(
            [pltpu.VMEM((n, n), jnp.bfloat16) for _ in range(4)]
            + [vec() for _ in range(5)]),
        compiler_params=pltpu.CompilerParams(
            dimension_semantics=("parallel",),
            vmem_limit_bytes=100 * 1024 * 1024,
        ),
    )(sc, gp, sw)

    x = feats[:, :, 0]

    d1, d2, d3, d4 = (p['fl1_w'].shape[1], p['fl2_w'].shape[1],
                      p['fl3_w'].shape[1], p['fl4_w'].shape[1])
    d4p = 128
    w4 = _pad_to(p['fl4_w'], (d3, d4p))
    c4 = jnp.full((d4p,), -1e30, jnp.float32).at[:d4].set(p['fl4_b'])

    args = [x,
            _pad_to(p['bn1_g'], (n,)).reshape(1, n),
            _pad_to(p['bn1_b'], (n,)).reshape(1, n),
            _pad_to(p['fl1_w'], (n, d1)), p['fl1_b'].reshape(1, d1),
            p['bn2_g'].reshape(1, -1), p['bn2_b'].reshape(1, -1),
            p['fl2_w'], p['fl2_b'].reshape(1, d2),
            p['bn3_g'].reshape(1, -1), p['bn3_b'].reshape(1, -1),
            p['fl3_w'], p['fl3_b'].reshape(1, d3),
            p['bn4_g'].reshape(1, -1), p['bn4_b'].reshape(1, -1),
            w4, c4.reshape(1, d4p)]

    probs = pl.pallas_call(
        _mlp_kernel,
        out_shape=jax.ShapeDtypeStruct((b, d4p), jnp.float32),
    )(*args)
    return probs[:, :d4]


# speedup vs baseline: 1.0413x; 1.0413x over previous
import functools

import jax
import jax.numpy as jnp
from jax.experimental import pallas as pl
from jax.experimental.pallas import tpu as pltpu

_EPS = 1e-8
_BLK = 128

_DW, _DB, _PW, _PB, _UW, _UB = 0, 4, 8, 12, 16, 20
_BW, _BB, _EW0, _EW1, _EB = 24, 25, 26, 27, 28


def _unet_kernel(sc_ref, gs_ref, sw_ref, out_ref,
                 u1, u2, u3, u4, yv, cv, sv, ov, mv, *, n_real, ks):
    uscr = [u1, u2, u3, u4]
    f32, bf16 = jnp.float32, jnp.bfloat16
    n = gs_ref.shape[1]
    nb = n // _BLK

    def loop(body):
        jax.lax.fori_loop(0, nb, lambda j, c: (body(j * _BLK), 0)[1], 0)

    def mv_gs(h):
        def body(r):
            yv[pl.ds(r, _BLK), :] = jnp.dot(
                gs_ref[0, pl.ds(r, _BLK), :], h, preferred_element_type=f32)
        loop(body)
        return yv[...]

    def mv_u(uref, h):
        def body(r):
            yv[pl.ds(r, _BLK), :] = jnp.dot(
                uref[pl.ds(r, _BLK), :].astype(f32), h,
                preferred_element_type=f32)
        loop(body)
        return yv[...]

    iota = jax.lax.broadcasted_iota(jnp.int32, (n, 1), 0)
    m = (iota < n_real).astype(f32)
    o = iota.astype(f32)

    h = jnp.maximum(mv_gs(sw_ref[...]), 0.0) * m
    org_h = h

    down_outs, saved, rs = [], [], None
    for i in range(4):
        if i == 0:
            y = mv_gs(h)
        else:
            y = mv_u(uscr[i - 1], h) / (rs + _EPS)
        h = jnp.maximum(y * sc_ref[0, _DW + i] + sc_ref[0, _DB + i], 0.0) * m
        down_outs.append(h)
        saved.append((None if i == 0 else i - 1, rs, m))

        s = jax.nn.sigmoid(h * sc_ref[0, _PW + i] + sc_ref[0, _PB + i])
        sv[...], ov[...], mv[...] = s, o, m
        s_r, o_r, m_r = jnp.transpose(s), jnp.transpose(o), jnp.transpose(m)

        def rank_body(r):
            sb = sv[pl.ds(r, _BLK), :]
            ob = ov[pl.ds(r, _BLK), :]
            better = (s_r > sb) | ((s_r == sb) & (o_r < ob))
            bet = jnp.where(better, 1.0, 0.0)
            cv[pl.ds(r, _BLK), :] = jnp.dot(bet, m,
                                            preferred_element_type=f32)
        loop(rank_body)
        cnt = cv[...]
        newm = m * (cnt < ks[i]).astype(f32)
        o = cnt
        h = h * s * newm

        if i == 0:
            rhs = (gs_ref[0] != 0).astype(bf16)
        else:
            rhs = uscr[i - 1][...]
        mv[...] = newm
        newm_r = jnp.transpose(newm)
        udst = uscr[i]
        one_f = jnp.ones((n, 1), f32)

        def sq_body(r, lvl=i, rhs=rhs, newm_r=newm_r, udst=udst,
                    one_f=one_f):
            if lvl == 0:
                lhs = (gs_ref[0, pl.ds(r, _BLK), :] != 0).astype(bf16)
            else:
                lhs = uscr[lvl - 1][pl.ds(r, _BLK), :]
            v = jnp.dot(lhs, rhs, preferred_element_type=f32)
            bmat = (v > 0).astype(f32) * mv[pl.ds(r, _BLK), :] * newm_r
            udst[pl.ds(r, _BLK), :] = bmat.astype(bf16)
            cv[pl.ds(r, _BLK), :] = jnp.dot(bmat, one_f,
                                            preferred_element_type=f32)
        loop(sq_body)
        rs = cv[...]
        m = newm

    y = mv_u(uscr[3], h) / (rs + _EPS)
    h = jnp.maximum(y * sc_ref[0, _BW] + sc_ref[0, _BB], 0.0) * m

    for i in range(4):
        up = 3 - i
        lvl, rsu, mu = saved[up]
        if lvl is None:
            y = mv_gs(h)
        else:
            y = mv_u(uscr[lvl], h) / (rsu + _EPS)
        h = jnp.maximum(y * sc_ref[0, _UW + i] + sc_ref[0, _UB + i], 0.0) * mu
        h = h + down_outs[up]

    y1 = mv_gs(h)
    y2 = mv_gs(org_h)
    out_ref[0] = jnp.maximum(
        y1 * sc_ref[0, _EW0] + y2 * sc_ref[0, _EW1] + sc_ref[0, _EB], 0.0)


def _mlp_kernel(x_ref, g1, b1, w1, c1, g2, b2, w2, c2, g3, b3, w3, c3,
                g4, b4, w4, c4, out_ref):
    def bn(x, ga, be):
        mu = jnp.mean(x, axis=0, keepdims=True)
        va = jnp.mean((x - mu) * (x - mu), axis=0, keepdims=True)
        return (x - mu) / jnp.sqrt(va + 1e-5) * ga[...] + be[...]

    f32 = jnp.float32
    h = x_ref[...]
    for ga, be, w, c in ((g1, b1, w1, c1), (g2, b2, w2, c2),
                         (g3, b3, w3, c3), (g4, b4, w4, c4)):
        h = jnp.maximum(bn(h, ga, be), 0.0)
        h = jnp.dot(h, w[...], preferred_element_type=f32) + c[...]
    zmax = jnp.max(h, axis=1, keepdims=True)
    e = jnp.exp(h - zmax)
    out_ref[...] = e / jnp.sum(e, axis=1, keepdims=True)


def _pad_to(x, shape):
    return jnp.pad(x, [(0, t - s) for s, t in zip(x.shape, shape)])


@jax.jit
def kernel(g, params):
    p = params
    b, n_real, _ = g.shape
    n = ((n_real + 127) // 128) * 128
    ks = [int(n_real * 4 / 5), int(n_real * 3 / 5),
          int(n_real * 2 / 5), int(n_real * 1 / 5)]

    gp = _pad_to(g.astype(jnp.float32), (b, n, n))
    sw = _pad_to(p['start_w'], (n, 1))

    sc = jnp.zeros((32,), jnp.float32)
    for i in range(4):
        sc = sc.at[_DW + i].set(p['down_w'][i][0, 0])
        sc = sc.at[_DB + i].set(p['down_b'][i][0])
        sc = sc.at[_PW + i].set(p['pool_w'][i][0, 0])
        sc = sc.at[_PB + i].set(p['pool_b'][i][0])
        sc = sc.at[_UW + i].set(p['up_w'][i][0, 0])
        sc = sc.at[_UB + i].set(p['up_b'][i][0])
    sc = sc.at[_BW].set(p['bottom_w'][0, 0]).at[_BB].set(p['bottom_b'][0])
    sc = sc.at[_EW0].set(p['end_w'][0, 0]).at[_EW1].set(p['end_w'][1, 0])
    sc = sc.at[_EB].set(p['end_b'][0])
    sc = sc.reshape(1, 32)

    vec = lambda: pltpu.VMEM((n, 1), jnp.float32)
    feats = pl.pallas_call(
        functools.partial(_unet_kernel, n_real=n_real, ks=ks),
        grid=(b,),
        in_specs=[
            pl.BlockSpec((1, 32), lambda i: (0, 0), memory_space=pltpu.SMEM),
            pl.BlockSpec((1, n, n), lambda i: (i, 0, 0)),
            pl.BlockSpec((n, 1), lambda i: (0, 0)),
        ],
        out_specs=pl.BlockSpec((1, n, 1), lambda i: (i, 0, 0)),
        out_shape=jax.ShapeDtypeStruct((b, n, 1), jnp.float32),
        scratch_shapes=(
            [pltpu.VMEM((n, n), jnp.bfloat16) for _ in range(4)]
            + [vec() for _ in range(5)]),
        compiler_params=pltpu.CompilerParams(
            dimension_semantics=("parallel",),
            vmem_limit_bytes=100 * 1024 * 1024,
        ),
    )(sc, gp, sw)

    x = feats[:, :, 0]

    d1, d2, d3, d4 = (p['fl1_w'].shape[1], p['fl2_w'].shape[1],
                      p['fl3_w'].shape[1], p['fl4_w'].shape[1])
    d4p = 128
    w4 = _pad_to(p['fl4_w'], (d3, d4p))
    c4 = jnp.full((d4p,), -1e30, jnp.float32).at[:d4].set(p['fl4_b'])

    args = [x,
            _pad_to(p['bn1_g'], (n,)).reshape(1, n),
            _pad_to(p['bn1_b'], (n,)).reshape(1, n),
            _pad_to(p['fl1_w'], (n, d1)), p['fl1_b'].reshape(1, d1),
            p['bn2_g'].reshape(1, -1), p['bn2_b'].reshape(1, -1),
            p['fl2_w'], p['fl2_b'].reshape(1, d2),
            p['bn3_g'].reshape(1, -1), p['bn3_b'].reshape(1, -1),
            p['fl3_w'], p['fl3_b'].reshape(1, d3),
            p['bn4_g'].reshape(1, -1), p['bn4_b'].reshape(1, -1),
            w4, c4.reshape(1, d4p)]

    probs = pl.pallas_call(
        _mlp_kernel,
        out_shape=jax.ShapeDtypeStruct((b, d4p), jnp.float32),
    )(*args)
    return probs[:, :d4]


# speedup vs baseline: 1.4023x; 1.3467x over previous
import functools

import jax
import jax.numpy as jnp
from jax.experimental import pallas as pl
from jax.experimental.pallas import tpu as pltpu

_EPS = 1e-8

_DW, _DB, _PW, _PB, _UW, _UB = 0, 4, 8, 12, 16, 20
_BW, _BB, _EW0, _EW1, _EB = 24, 25, 26, 27, 28


def _unet_kernel(sc_ref, gs_ref, sw_ref, out_ref,
                 u1, u2, u3, u4, yv, cv, sv, ov, mv, *, n_real, ks, blk):
    uscr = [u1, u2, u3, u4]
    f32, bf16 = jnp.float32, jnp.bfloat16
    n = gs_ref.shape[1]
    nb = n // blk

    def loop(body):
        jax.lax.fori_loop(0, nb, lambda j, c: (body(j * blk), 0)[1], 0)

    def mv_gs(h):
        def body(r):
            yv[pl.ds(r, blk), :] = jnp.dot(
                gs_ref[0, pl.ds(r, blk), :], h, preferred_element_type=f32)
        loop(body)
        return yv[...]

    def mv_u(uref, h):
        def body(r):
            yv[pl.ds(r, blk), :] = jnp.dot(
                uref[pl.ds(r, blk), :].astype(f32), h,
                preferred_element_type=f32)
        loop(body)
        return yv[...]

    iota = jax.lax.broadcasted_iota(jnp.int32, (n, 1), 0)
    m = (iota < n_real).astype(f32)
    o = iota.astype(f32)

    h = jnp.maximum(mv_gs(sw_ref[...]), 0.0) * m
    org_h = h

    down_outs, saved, rs = [], [], None
    for i in range(4):
        if i == 0:
            y = mv_gs(h)
        else:
            y = mv_u(uscr[i - 1], h) / (rs + _EPS)
        h = jnp.maximum(y * sc_ref[0, _DW + i] + sc_ref[0, _DB + i], 0.0) * m
        down_outs.append(h)
        saved.append((None if i == 0 else i - 1, rs, m))

        s = jax.nn.sigmoid(h * sc_ref[0, _PW + i] + sc_ref[0, _PB + i])
        sv[...], ov[...], mv[...] = s, o, m
        s_r, o_r, m_r = jnp.transpose(s), jnp.transpose(o), jnp.transpose(m)

        def rank_body(r):
            sb = sv[pl.ds(r, blk), :]
            ob = ov[pl.ds(r, blk), :]
            better = (s_r > sb) | ((s_r == sb) & (o_r < ob))
            cv[pl.ds(r, blk), :] = jnp.sum(
                jnp.where(better, m_r, 0.0), axis=1, keepdims=True)
        loop(rank_body)
        cnt = cv[...]
        newm = m * (cnt < ks[i]).astype(f32)
        o = cnt
        h = h * s * newm

        if i == 0:
            rhs = (gs_ref[0] != 0).astype(bf16)
        else:
            rhs = uscr[i - 1][...]
        mv[...] = newm
        newm_r = jnp.transpose(newm)
        udst = uscr[i]

        def sq_body(r, lvl=i, rhs=rhs, newm_r=newm_r, udst=udst):
            if lvl == 0:
                lhs = (gs_ref[0, pl.ds(r, blk), :] != 0).astype(bf16)
            else:
                lhs = uscr[lvl - 1][pl.ds(r, blk), :]
            v = jnp.dot(lhs, rhs, preferred_element_type=f32)
            bmat = (v > 0).astype(f32) * mv[pl.ds(r, blk), :] * newm_r
            cv[pl.ds(r, blk), :] = jnp.sum(bmat, axis=1, keepdims=True)
            udst[pl.ds(r, blk), :] = bmat.astype(bf16)
        loop(sq_body)
        rs = cv[...]
        m = newm

    y = mv_u(uscr[3], h) / (rs + _EPS)
    h = jnp.maximum(y * sc_ref[0, _BW] + sc_ref[0, _BB], 0.0) * m

    for i in range(4):
        up = 3 - i
        lvl, rsu, mu = saved[up]
        if lvl is None:
            y = mv_gs(h)
        else:
            y = mv_u(uscr[lvl], h) / (rsu + _EPS)
        h = jnp.maximum(y * sc_ref[0, _UW + i] + sc_ref[0, _UB + i], 0.0) * mu
        h = h + down_outs[up]

    y1 = mv_gs(h)
    y2 = mv_gs(org_h)
    out_ref[0] = jnp.maximum(
        y1 * sc_ref[0, _EW0] + y2 * sc_ref[0, _EW1] + sc_ref[0, _EB], 0.0)


def _mlp_kernel(x_ref, g1, b1, w1, c1, g2, b2, w2, c2, g3, b3, w3, c3,
                g4, b4, w4, c4, out_ref):
    def bn(x, ga, be):
        mu = jnp.mean(x, axis=0, keepdims=True)
        va = jnp.mean((x - mu) * (x - mu), axis=0, keepdims=True)
        return (x - mu) / jnp.sqrt(va + 1e-5) * ga[...] + be[...]

    f32 = jnp.float32
    h = x_ref[...]
    for ga, be, w, c in ((g1, b1, w1, c1), (g2, b2, w2, c2),
                         (g3, b3, w3, c3), (g4, b4, w4, c4)):
        h = jnp.maximum(bn(h, ga, be), 0.0)
        h = jnp.dot(h, w[...], preferred_element_type=f32) + c[...]
    zmax = jnp.max(h, axis=1, keepdims=True)
    e = jnp.exp(h - zmax)
    out_ref[...] = e / jnp.sum(e, axis=1, keepdims=True)


def _pad_to(x, shape):
    return jnp.pad(x, [(0, t - s) for s, t in zip(x.shape, shape)])


@jax.jit
def kernel(g, params):
    p = params
    b, n_real, _ = g.shape
    n = ((n_real + 127) // 128) * 128
    ks = [int(n_real * 4 / 5), int(n_real * 3 / 5),
          int(n_real * 2 / 5), int(n_real * 1 / 5)]

    gp = _pad_to(g.astype(jnp.float32), (b, n, n))
    sw = _pad_to(p['start_w'], (n, 1))

    sc = jnp.zeros((32,), jnp.float32)
    for i in range(4):
        sc = sc.at[_DW + i].set(p['down_w'][i][0, 0])
        sc = sc.at[_DB + i].set(p['down_b'][i][0])
        sc = sc.at[_PW + i].set(p['pool_w'][i][0, 0])
        sc = sc.at[_PB + i].set(p['pool_b'][i][0])
        sc = sc.at[_UW + i].set(p['up_w'][i][0, 0])
        sc = sc.at[_UB + i].set(p['up_b'][i][0])
    sc = sc.at[_BW].set(p['bottom_w'][0, 0]).at[_BB].set(p['bottom_b'][0])
    sc = sc.at[_EW0].set(p['end_w'][0, 0]).at[_EW1].set(p['end_w'][1, 0])
    sc = sc.at[_EB].set(p['end_b'][0])
    sc = sc.reshape(1, 32)

    vec = lambda: pltpu.VMEM((n, 1), jnp.float32)
    feats = pl.pallas_call(
        functools.partial(_unet_kernel, n_real=n_real, ks=ks,
                          blk=next(bs for bs in (384, 256, 128) if n % bs == 0)),
        grid=(b,),
        in_specs=[
            pl.BlockSpec((1, 32), lambda i: (0, 0), memory_space=pltpu.SMEM),
            pl.BlockSpec((1, n, n), lambda i: (i, 0, 0)),
            pl.BlockSpec((n, 1), lambda i: (0, 0)),
        ],
        out_specs=pl.BlockSpec((1, n, 1), lambda i: (i, 0, 0)),
        out_shape=jax.ShapeDtypeStruct((b, n, 1), jnp.float32),
        scratch_shapes=(
            [pltpu.VMEM((n, n), jnp.bfloat16) for _ in range(4)]
            + [vec() for _ in range(5)]),
        compiler_params=pltpu.CompilerParams(
            dimension_semantics=("parallel",),
            vmem_limit_bytes=100 * 1024 * 1024,
        ),
    )(sc, gp, sw)

    x = feats[:, :, 0]

    d1, d2, d3, d4 = (p['fl1_w'].shape[1], p['fl2_w'].shape[1],
                      p['fl3_w'].shape[1], p['fl4_w'].shape[1])
    d4p = 128
    w4 = _pad_to(p['fl4_w'], (d3, d4p))
    c4 = jnp.full((d4p,), -1e30, jnp.float32).at[:d4].set(p['fl4_b'])

    args = [x,
            _pad_to(p['bn1_g'], (n,)).reshape(1, n),
            _pad_to(p['bn1_b'], (n,)).reshape(1, n),
            _pad_to(p['fl1_w'], (n, d1)), p['fl1_b'].reshape(1, d1),
            p['bn2_g'].reshape(1, -1), p['bn2_b'].reshape(1, -1),
            p['fl2_w'], p['fl2_b'].reshape(1, d2),
            p['bn3_g'].reshape(1, -1), p['bn3_b'].reshape(1, -1),
            p['fl3_w'], p['fl3_b'].reshape(1, d3),
            p['bn4_g'].reshape(1, -1), p['bn4_b'].reshape(1, -1),
            w4, c4.reshape(1, d4p)]

    probs = pl.pallas_call(
        _mlp_kernel,
        out_shape=jax.ShapeDtypeStruct((b, d4p), jnp.float32),
    )(*args)
    return probs[:, :d4]


# speedup vs baseline: 1.5321x; 1.0926x over previous
import functools

import jax
import jax.numpy as jnp
from jax.experimental import pallas as pl
from jax.experimental.pallas import tpu as pltpu

_EPS = 1e-8

_DW, _DB, _PW, _PB, _UW, _UB = 0, 4, 8, 12, 16, 20
_BW, _BB, _EW0, _EW1, _EB = 24, 25, 26, 27, 28


def _unet_kernel(sc_ref, gs_ref, sw_ref, out_ref,
                 u1, u2, u3, u4, yv, cv, sv, ov, mv, *, n_real, ks, blk):
    uscr = [u1, u2, u3, u4]
    f32, bf16 = jnp.float32, jnp.bfloat16
    n = gs_ref.shape[1]
    nb = n // blk

    def loop(body):
        jax.lax.fori_loop(0, nb, lambda j, c: (body(j * blk), 0)[1], 0)

    def mv_gs(h):
        def body(r):
            yv[pl.ds(r, blk), :] = jnp.dot(
                gs_ref[0, pl.ds(r, blk), :], h, preferred_element_type=f32)
        loop(body)
        return yv[...]

    def mv_u(uref, h):
        def body(r):
            yv[pl.ds(r, blk), :] = jnp.dot(
                uref[pl.ds(r, blk), :].astype(f32), h,
                preferred_element_type=f32)
        loop(body)
        return yv[...]

    iota = jax.lax.broadcasted_iota(jnp.int32, (n, 1), 0)
    m = (iota < n_real).astype(f32)
    o = iota.astype(f32)

    h = jnp.maximum(mv_gs(sw_ref[...]), 0.0) * m
    org_h = h

    down_outs, saved, rs = [], [], None
    for i in range(4):
        if i == 0:
            y = mv_gs(h)
        else:
            y = mv_u(uscr[i - 1], h) / (rs + _EPS)
        h = jnp.maximum(y * sc_ref[0, _DW + i] + sc_ref[0, _DB + i], 0.0) * m
        down_outs.append(h)
        saved.append((None if i == 0 else i - 1, rs, m))

        s = jax.nn.sigmoid(h * sc_ref[0, _PW + i] + sc_ref[0, _PB + i])
        sv[...], ov[...], mv[...] = s, o, m
        s_r, o_r, m_r = jnp.transpose(s), jnp.transpose(o), jnp.transpose(m)

        def rank_body(r):
            sb = sv[pl.ds(r, blk), :]
            ob = ov[pl.ds(r, blk), :]
            better = (s_r > sb) | ((s_r == sb) & (o_r < ob))
            cv[pl.ds(r, blk), :] = jnp.sum(
                jnp.where(better, m_r, 0.0), axis=1, keepdims=True)
        loop(rank_body)
        cnt = cv[...]
        newm = m * (cnt < ks[i]).astype(f32)
        o = cnt
        h = h * s * newm

        if i == 0:
            rhs = (gs_ref[0] != 0).astype(bf16)
        else:
            rhs = uscr[i - 1][...]
        mv[...] = newm
        newm_r = jnp.transpose(newm)
        udst = uscr[i]

        def sq_body(r, lvl=i, rhs=rhs, newm_r=newm_r, udst=udst):
            if lvl == 0:
                lhs = (gs_ref[0, pl.ds(r, blk), :] != 0).astype(bf16)
            else:
                lhs = uscr[lvl - 1][pl.ds(r, blk), :]
            v = jnp.dot(lhs, rhs, preferred_element_type=f32)
            bmat = (v > 0).astype(f32) * mv[pl.ds(r, blk), :] * newm_r
            cv[pl.ds(r, blk), :] = jnp.sum(bmat, axis=1, keepdims=True)
            udst[pl.ds(r, blk), :] = bmat.astype(bf16)
        loop(sq_body)
        rs = cv[...]
        m = newm

    y = mv_u(uscr[3], h) / (rs + _EPS)
    h = jnp.maximum(y * sc_ref[0, _BW] + sc_ref[0, _BB], 0.0) * m

    for i in range(4):
        up = 3 - i
        lvl, rsu, mu = saved[up]
        if lvl is None:
            y = mv_gs(h)
        else:
            y = mv_u(uscr[lvl], h) / (rsu + _EPS)
        h = jnp.maximum(y * sc_ref[0, _UW + i] + sc_ref[0, _UB + i], 0.0) * mu
        h = h + down_outs[up]

    y1 = mv_gs(h)
    y2 = mv_gs(org_h)
    out_ref[0] = jnp.maximum(
        y1 * sc_ref[0, _EW0] + y2 * sc_ref[0, _EW1] + sc_ref[0, _EB], 0.0)


def _mlp_kernel(x_ref, g1, b1, w1, c1, g2, b2, w2, c2, g3, b3, w3, c3,
                g4, b4, w4, c4, out_ref):
    def bn(x, ga, be):
        mu = jnp.mean(x, axis=0, keepdims=True)
        va = jnp.mean((x - mu) * (x - mu), axis=0, keepdims=True)
        return (x - mu) / jnp.sqrt(va + 1e-5) * ga[...] + be[...]

    f32 = jnp.float32
    h = x_ref[...]
    for ga, be, w, c in ((g1, b1, w1, c1), (g2, b2, w2, c2),
                         (g3, b3, w3, c3), (g4, b4, w4, c4)):
        h = jnp.maximum(bn(h, ga, be), 0.0)
        h = jnp.dot(h, w[...], preferred_element_type=f32) + c[...]
    zmax = jnp.max(h, axis=1, keepdims=True)
    e = jnp.exp(h - zmax)
    out_ref[...] = e / jnp.sum(e, axis=1, keepdims=True)


def _pad_to(x, shape):
    return jnp.pad(x, [(0, t - s) for s, t in zip(x.shape, shape)])


@jax.jit
def kernel(g, params):
    p = params
    b, n_real, _ = g.shape
    n = ((n_real + 127) // 128) * 128
    ks = [int(n_real * 4 / 5), int(n_real * 3 / 5),
          int(n_real * 2 / 5), int(n_real * 1 / 5)]

    gp = _pad_to(g.astype(jnp.float32), (b, n, n))
    sw = _pad_to(p['start_w'], (n, 1))

    sc = jnp.zeros((32,), jnp.float32)
    for i in range(4):
        sc = sc.at[_DW + i].set(p['down_w'][i][0, 0])
        sc = sc.at[_DB + i].set(p['down_b'][i][0])
        sc = sc.at[_PW + i].set(p['pool_w'][i][0, 0])
        sc = sc.at[_PB + i].set(p['pool_b'][i][0])
        sc = sc.at[_UW + i].set(p['up_w'][i][0, 0])
        sc = sc.at[_UB + i].set(p['up_b'][i][0])
    sc = sc.at[_BW].set(p['bottom_w'][0, 0]).at[_BB].set(p['bottom_b'][0])
    sc = sc.at[_EW0].set(p['end_w'][0, 0]).at[_EW1].set(p['end_w'][1, 0])
    sc = sc.at[_EB].set(p['end_b'][0])
    sc = sc.reshape(1, 32)

    vec = lambda: pltpu.VMEM((n, 1), jnp.float32)
    feats = pl.pallas_call(
        functools.partial(_unet_kernel, n_real=n_real, ks=ks,
                          blk=next(bs for bs in (768, 512, 384, 256, 128)
                                   if n % bs == 0)),
        grid=(b,),
        in_specs=[
            pl.BlockSpec((1, 32), lambda i: (0, 0), memory_space=pltpu.SMEM),
            pl.BlockSpec((1, n, n), lambda i: (i, 0, 0)),
            pl.BlockSpec((n, 1), lambda i: (0, 0)),
        ],
        out_specs=pl.BlockSpec((1, n, 1), lambda i: (i, 0, 0)),
        out_shape=jax.ShapeDtypeStruct((b, n, 1), jnp.float32),
        scratch_shapes=(
            [pltpu.VMEM((n, n), jnp.bfloat16) for _ in range(4)]
            + [vec() for _ in range(5)]),
        compiler_params=pltpu.CompilerParams(
            dimension_semantics=("parallel",),
            vmem_limit_bytes=100 * 1024 * 1024,
        ),
    )(sc, gp, sw)

    x = feats[:, :, 0]

    d1, d2, d3, d4 = (p['fl1_w'].shape[1], p['fl2_w'].shape[1],
                      p['fl3_w'].shape[1], p['fl4_w'].shape[1])
    d4p = 128
    w4 = _pad_to(p['fl4_w'], (d3, d4p))
    c4 = jnp.full((d4p,), -1e30, jnp.float32).at[:d4].set(p['fl4_b'])

    args = [x,
            _pad_to(p['bn1_g'], (n,)).reshape(1, n),
            _pad_to(p['bn1_b'], (n,)).reshape(1, n),
            _pad_to(p['fl1_w'], (n, d1)), p['fl1_b'].reshape(1, d1),
            p['bn2_g'].reshape(1, -1), p['bn2_b'].reshape(1, -1),
            p['fl2_w'], p['fl2_b'].reshape(1, d2),
            p['bn3_g'].reshape(1, -1), p['bn3_b'].reshape(1, -1),
            p['fl3_w'], p['fl3_b'].reshape(1, d3),
            p['bn4_g'].reshape(1, -1), p['bn4_b'].reshape(1, -1),
            w4, c4.reshape(1, d4p)]

    probs = pl.pallas_call(
        _mlp_kernel,
        out_shape=jax.ShapeDtypeStruct((b, d4p), jnp.float32),
    )(*args)
    return probs[:, :d4]
